# baseline (speedup 1.0000x reference)
import functools

import jax
import jax.numpy as jnp
from jax import lax
from jax.experimental import pallas as pl
from jax.experimental.pallas import tpu as pltpu
from jax.experimental.pallas import tpu_sc as plsc

DIV = 1000
EMB_DIM = 128
L = 16

_info = plsc.get_sparse_core_info()
NC, NS = _info.num_cores, _info.num_subcores
NW = NC * NS

IDX_C = 128
GROUP = 16


def _pack_table(emb):
    eb = emb.astype(jnp.bfloat16)
    u = lax.bitcast_convert_type(eb, jnp.uint16).astype(jnp.uint32)
    u2 = u.reshape(emb.shape[0], 2, 64)
    w = u2[:, 0, :] | (u2[:, 1, :] << 16)
    return lax.bitcast_convert_type(w, jnp.int32).reshape(emb.shape[0] * 64)


def _rsqrt_f32(s):
    ib = lax.bitcast_convert_type(s, jnp.int32)
    yi = jnp.int32(0x5F3759DF) - lax.shift_right_logical(ib, 1)
    y = lax.bitcast_convert_type(yi, jnp.float32)
    for _ in range(2):
        y = y * (1.5 - 0.5 * s * y * y)
    return y


def _unpack_lo(w32):
    return lax.bitcast_convert_type(lax.shift_left(w32, 16), jnp.float32)


def _unpack_hi(w32):
    return lax.bitcast_convert_type(w32 & jnp.int32(-65536), jnp.float32)


@functools.partial(jax.jit, static_argnames=("n_total",))
def _qr_embed_sc(x_flat, qtab, rtab, *, n_total):
    rows_per_w = n_total // NW
    n_blocks = rows_per_w // IDX_C
    mesh = plsc.VectorSubcoreMesh(core_axis_name="c", subcore_axis_name="s")

    @functools.partial(
        pl.kernel,
        out_type=jax.ShapeDtypeStruct((n_total * EMB_DIM,), jnp.float32),
        mesh=mesh,
        compiler_params=pltpu.CompilerParams(needs_layout_passes=False),
        scratch_types=[
            pltpu.VMEM((DIV * 64,), jnp.int32),
            pltpu.VMEM((DIV * 64,), jnp.int32),
            pltpu.VMEM((IDX_C,), jnp.int32),
            pltpu.VMEM((IDX_C,), jnp.int32),
            pltpu.VMEM((GROUP * EMB_DIM,), jnp.float32),
            pltpu.SemaphoreType.DMA,
        ],
    )
    def body(x_hbm, qtab_hbm, rtab_hbm, out_hbm,
             qtab_v, rtab_v, idxq_v, idxr_v, out_v, sem):
        wid = lax.axis_index("s") * NC + lax.axis_index("c")
        pltpu.sync_copy(qtab_hbm, qtab_v)
        pltpu.sync_copy(rtab_hbm, rtab_v)
        lane = lax.broadcasted_iota(jnp.int32, (L,), 0)
        obase = lane * EMB_DIM
        obase64 = obase + 64

        def block_body(bi, carry):
            bbase = wid * rows_per_w + bi * IDX_C
            pltpu.sync_copy(x_hbm.at[pl.ds(bbase, IDX_C)], idxq_v)

            def qr_body(j, carry2):
                v = idxq_v[pl.ds(j * L, L)]
                qv = lax.div(v, jnp.int32(DIV))
                idxq_v[pl.ds(j * L, L)] = qv * 64
                idxr_v[pl.ds(j * L, L)] = (v - qv * jnp.int32(DIV)) * 64
                return carry2

            lax.fori_loop(0, IDX_C // L, qr_body, 0)

            def group_body(g, carry3):
                q64 = idxq_v[pl.ds(g * L, L)]
                r64 = idxr_v[pl.ds(g * L, L)]
                dst0 = out_hbm.at[
                    pl.ds((bbase + g * L) * EMB_DIM, GROUP * EMB_DIM)]

                @pl.when(bi * (IDX_C // L) + g > 0)
                def _wait_prev0():
                    pltpu.make_async_copy(out_v, dst0, sem).wait()

                zero2 = jnp.zeros((2 * L,), jnp.bfloat16)

                @plsc.parallel_loop(0, 64, step=2, unroll=8,
                                    carry=(zero2, zero2))
                def p1_acc(w, accs):
                    a0, a1 = accs
                    for u in range(2):
                        wl = (w + u + lane) & jnp.int32(63)
                        wq = plsc.load_gather(qtab_v, [q64 + wl])
                        wr = plsc.load_gather(rtab_v, [r64 + wl])
                        s = (plsc.bitcast(wq, jnp.bfloat16)
                             + plsc.bitcast(wr, jnp.bfloat16))
                        if u == 0:
                            a0 = a0 + s * s
                        else:
                            a1 = a1 + s * s
                    return a0, a1

                a0, a1 = p1_acc
                acc = plsc.bitcast(a0 + a1, jnp.int32)
                tot = _unpack_lo(acc) + _unpack_hi(acc)
                inv = _rsqrt_f32(jnp.maximum(tot, jnp.float32(1e-24)))
                dst = dst0
                @plsc.parallel_loop(0, 64, step=1, unroll=16)
                def p2_body(w):
                    wl = (w + lane) & jnp.int32(63)
                    wq = plsc.load_gather(qtab_v, [q64 + wl])
                    wr = plsc.load_gather(rtab_v, [r64 + wl])
                    s = plsc.bitcast(plsc.bitcast(wq, jnp.bfloat16)
                                     + plsc.bitcast(wr, jnp.bfloat16),
                                     jnp.int32)
                    plsc.store_scatter(out_v, [obase + wl],
                                       _unpack_lo(s) * inv)
                    plsc.store_scatter(out_v, [obase64 + wl],
                                       _unpack_hi(s) * inv)
                pltpu.async_copy(out_v, dst, sem)
                return carry3

            lax.fori_loop(0, IDX_C // L, group_body, 0)
            return carry

        lax.fori_loop(0, n_blocks, block_body, 0)
        tail = out_hbm.at[
            pl.ds(wid * rows_per_w * EMB_DIM, GROUP * EMB_DIM)]
        pltpu.make_async_copy(out_v, tail, sem).wait()

    return body(x_flat, qtab, rtab)


def kernel(x, emb_q, emb_r):
    n_total = x.shape[0] * x.shape[1]
    x_flat = x.reshape(n_total)
    out = _qr_embed_sc(x_flat, _pack_table(emb_q), _pack_table(emb_r),
                       n_total=n_total)
    return out.reshape(x.shape[0], x.shape[1], EMB_DIM)


_ = pl.pallas_call


# speedup vs baseline: 1.1063x; 1.1063x over previous
import functools

import jax
import jax.numpy as jnp
from jax import lax
from jax.experimental import pallas as pl
from jax.experimental.pallas import tpu as pltpu
from jax.experimental.pallas import tpu_sc as plsc

DIV = 1000
EMB_DIM = 128
L = 16

_info = plsc.get_sparse_core_info()
NC, NS = _info.num_cores, _info.num_subcores
NW = NC * NS

IDX_C = 256
GROUP = 16


def _pack_table(emb):
    eb = emb.astype(jnp.bfloat16)
    u = lax.bitcast_convert_type(eb, jnp.uint16).astype(jnp.uint32)
    u2 = u.reshape(emb.shape[0], 2, 64)
    w = u2[:, 0, :] | (u2[:, 1, :] << 16)
    return lax.bitcast_convert_type(w, jnp.int32).reshape(emb.shape[0] * 64)


def _rsqrt_f32(s):
    ib = lax.bitcast_convert_type(s, jnp.int32)
    yi = jnp.int32(0x5F3759DF) - lax.shift_right_logical(ib, 1)
    y = lax.bitcast_convert_type(yi, jnp.float32)
    for _ in range(2):
        y = y * (1.5 - 0.5 * s * y * y)
    return y


def _unpack_lo(w32):
    return lax.bitcast_convert_type(lax.shift_left(w32, 16), jnp.float32)


def _unpack_hi(w32):
    return lax.bitcast_convert_type(w32 & jnp.int32(-65536), jnp.float32)


@functools.partial(jax.jit, static_argnames=("n_total",))
def _qr_embed_sc(x_flat, qtab, rtab, *, n_total):
    rows_per_w = n_total // NW
    n_blocks = rows_per_w // IDX_C
    mesh = plsc.VectorSubcoreMesh(core_axis_name="c", subcore_axis_name="s")

    @functools.partial(
        pl.kernel,
        out_type=jax.ShapeDtypeStruct((n_total * EMB_DIM,), jnp.float32),
        mesh=mesh,
        compiler_params=pltpu.CompilerParams(needs_layout_passes=False),
        scratch_types=[
            pltpu.VMEM((DIV * 64,), jnp.int32),
            pltpu.VMEM((DIV * 64,), jnp.int32),
            pltpu.VMEM((IDX_C,), jnp.int32),
            pltpu.VMEM((IDX_C,), jnp.int32),
            pltpu.VMEM((GROUP * EMB_DIM,), jnp.float32),
            pltpu.SemaphoreType.DMA,
        ],
    )
    def body(x_hbm, qtab_hbm, rtab_hbm, out_hbm,
             qtab_v, rtab_v, idxq_v, idxr_v, out_v, sem):
        wid = lax.axis_index("s") * NC + lax.axis_index("c")
        pltpu.sync_copy(qtab_hbm, qtab_v)
        pltpu.sync_copy(rtab_hbm, rtab_v)
        lane = lax.broadcasted_iota(jnp.int32, (L,), 0)
        obase = lane * EMB_DIM
        obase64 = obase + 64

        def block_body(bi, carry):
            bbase = wid * rows_per_w + bi * IDX_C
            pltpu.sync_copy(x_hbm.at[pl.ds(bbase, IDX_C)], idxq_v)

            def qr_body(j, carry2):
                v = idxq_v[pl.ds(j * L, L)]
                qv = lax.div(v, jnp.int32(DIV))
                idxq_v[pl.ds(j * L, L)] = qv * 64
                idxr_v[pl.ds(j * L, L)] = (v - qv * jnp.int32(DIV)) * 64
                return carry2

            lax.fori_loop(0, IDX_C // L, qr_body, 0)

            def group_body(g, carry3):
                q64 = idxq_v[pl.ds(g * L, L)]
                r64 = idxr_v[pl.ds(g * L, L)]
                zero2 = jnp.zeros((2 * L,), jnp.bfloat16)

                @plsc.parallel_loop(0, 64, step=2, unroll=8,
                                    carry=(zero2, zero2))
                def p1_acc(w, accs):
                    a0, a1 = accs
                    for u in range(2):
                        wl = (w + u + lane) & jnp.int32(63)
                        wq = plsc.load_gather(qtab_v, [q64 + wl])
                        wr = plsc.load_gather(rtab_v, [r64 + wl])
                        s = (plsc.bitcast(wq, jnp.bfloat16)
                             + plsc.bitcast(wr, jnp.bfloat16))
                        if u == 0:
                            a0 = a0 + s * s
                        else:
                            a1 = a1 + s * s
                    return a0, a1

                a0, a1 = p1_acc
                acc = plsc.bitcast(a0 + a1, jnp.int32)
                tot = _unpack_lo(acc) + _unpack_hi(acc)
                inv = _rsqrt_f32(jnp.maximum(tot, jnp.float32(1e-24)))
                dst = out_hbm.at[
                    pl.ds((bbase + g * L) * EMB_DIM, GROUP * EMB_DIM)]

                @pl.when(bi * (IDX_C // L) + g > 0)
                def _wait_prev():
                    pltpu.make_async_copy(out_v, dst, sem).wait()

                @plsc.parallel_loop(0, 64, step=1, unroll=16)
                def p2_body(w):
                    wl = (w + lane) & jnp.int32(63)
                    wq = plsc.load_gather(qtab_v, [q64 + wl])
                    wr = plsc.load_gather(rtab_v, [r64 + wl])
                    s = plsc.bitcast(plsc.bitcast(wq, jnp.bfloat16)
                                     + plsc.bitcast(wr, jnp.bfloat16),
                                     jnp.int32)
                    plsc.store_scatter(out_v, [obase + wl],
                                       _unpack_lo(s) * inv)
                    plsc.store_scatter(out_v, [obase64 + wl],
                                       _unpack_hi(s) * inv)
                pltpu.async_copy(out_v, dst, sem)
                return carry3

            lax.fori_loop(0, IDX_C // L, group_body, 0)
            return carry

        lax.fori_loop(0, n_blocks, block_body, 0)
        tail = out_hbm.at[
            pl.ds(wid * rows_per_w * EMB_DIM, GROUP * EMB_DIM)]
        pltpu.make_async_copy(out_v, tail, sem).wait()

    return body(x_flat, qtab, rtab)


def kernel(x, emb_q, emb_r):
    n_total = x.shape[0] * x.shape[1]
    x_flat = x.reshape(n_total)
    out = _qr_embed_sc(x_flat, _pack_table(emb_q), _pack_table(emb_r),
                       n_total=n_total)
    return out.reshape(x.shape[0], x.shape[1], EMB_DIM)


_ = pl.pallas_call
